# baseline (speedup 1.0000x reference)
import jax
import jax.numpy as jnp
from jax import lax
from jax.experimental import pallas as pl
from jax.experimental.pallas import tpu as pltpu

N = 10000
E = 160000
M = 6
H = 8
D = 128
DH = D // 2
HD = D // H

BN = 80
NB = N // BN
CHUNK = 512
E_PAD = ((E + CHUNK - 1) // CHUNK) * CHUNK


def _layer_kernel(rs_ref, ego_ref, ef_hbm, nf_hbm, bt_hbm,
                  Wq_r, bq_r, Wk_r, bk_r, Wv1_r, bv1_r, Wv2_r, bv2_r,
                  g1_r, be1_r, g2_r, be2_r, W1_r, b1_r, Wd_r, bd_r, W2_r, b2_r,
                  out_ref,
                  ef_v, nf_v, bt_v, m_s, s_s, acc, sem_ef, sem_nf, sem_bt):
    b = pl.program_id(0)
    n0 = b * BN
    s0 = rs_ref[b]
    s1 = rs_ref[b + 1]
    s0_al = (s0 // CHUNK) * CHUNK
    nchunks = (s1 - s0_al + CHUNK - 1) // CHUNK

    r_i = lax.broadcasted_iota(jnp.int32, (D, H), 0)
    c_i = lax.broadcasted_iota(jnp.int32, (D, H), 1)
    G = ((r_i // HD) == c_i).astype(jnp.float32)
    GT = G.T

    x = ego_ref[...]
    x2 = x.reshape(BN * M, D)
    mu = jnp.mean(x2, axis=-1, keepdims=True)
    var = jnp.mean((x2 - mu) ** 2, axis=-1, keepdims=True)
    ln1 = (x2 - mu) / jnp.sqrt(var + 1e-5) * g1_r[...] + be1_r[...]
    q = ln1 @ Wq_r[...] + bq_r[...]
    q3 = q.reshape(BN, M, D)

    m_s[...] = jnp.full((M, BN, H), -1e9, jnp.float32)
    s_s[...] = jnp.zeros((M, BN, H), jnp.float32)
    acc[...] = jnp.zeros((M, BN, D), jnp.float32)

    col = lax.broadcasted_iota(jnp.int32, (CHUNK, BN), 1)
    row1 = lax.broadcasted_iota(jnp.int32, (CHUNK, 1), 0)

    def start(c, slot):
        off = s0_al + c * CHUNK
        pltpu.make_async_copy(
            ef_hbm.at[pl.ds(off, CHUNK)], ef_v.at[slot], sem_ef.at[slot]
        ).start()
        pltpu.make_async_copy(
            nf_hbm.at[pl.ds(off, CHUNK)], nf_v.at[slot], sem_nf.at[slot]
        ).start()
        pltpu.make_async_copy(
            bt_hbm.at[pl.ds(off, CHUNK)], bt_v.at[slot], sem_bt.at[slot]
        ).start()

    def wait(c, slot):
        off = s0_al + c * CHUNK
        pltpu.make_async_copy(
            ef_hbm.at[pl.ds(off, CHUNK)], ef_v.at[slot], sem_ef.at[slot]
        ).wait()
        pltpu.make_async_copy(
            nf_hbm.at[pl.ds(off, CHUNK)], nf_v.at[slot], sem_nf.at[slot]
        ).wait()
        pltpu.make_async_copy(
            bt_hbm.at[pl.ds(off, CHUNK)], bt_v.at[slot], sem_bt.at[slot]
        ).wait()

    def compute(c, slot):
        off = s0_al + c * CHUNK
        ef3 = ef_v[slot]
        nf3 = nf_v[slot]
        bt = bt_v[slot].reshape(CHUNK, 1)
        li = bt - n0
        g = off + row1
        valid = (g >= s0) & (g < s1)
        onehot = jnp.where((li == col) & valid, 1.0, 0.0)
        for m in range(M):
            km = ef3[:, m, :] @ Wk_r[...] + bk_r[...]
            qm = onehot @ q3[:, m, :]
            lg = (km * qm) @ G
            lgm = jnp.where(valid, lg, -1e9)
            cand = jnp.max(
                jnp.where(onehot[:, :, None] > 0.0, lgm[:, None, :], -1e9),
                axis=0)
            m_old = m_s[m]
            m_new = jnp.maximum(m_old, cand)
            m_s[m] = m_new
            r = jnp.exp(m_old - m_new)
            e = jnp.where(valid, jnp.exp(lg - (onehot @ m_new)), 0.0)
            s_s[m] = s_s[m] * r + lax.dot_general(
                onehot, e, (((0,), (0,)), ((), ())))
            vm = (ef3[:, m, :] @ Wv1_r[...] + bv1_r[...]
                  + nf3[:, m, :] @ Wv2_r[...] + bv2_r[...])
            w = vm * (e @ GT)
            acc[m] = acc[m] * (r @ GT) + lax.dot_general(
                onehot, w, (((0,), (0,)), ((), ())))

    @pl.when(nchunks > 0)
    def _prologue():
        start(0, 0)

    def body2(i, carry):
        c0 = 2 * i

        @pl.when(c0 + 1 < nchunks)
        def _():
            start(c0 + 1, 1)

        wait(c0, 0)
        compute(c0, 0)

        @pl.when(c0 + 2 < nchunks)
        def _():
            start(c0 + 2, 0)

        @pl.when(c0 + 1 < nchunks)
        def _():
            wait(c0 + 1, 1)
            compute(c0 + 1, 1)

        return carry

    lax.fori_loop(0, (nchunks + 1) // 2, body2, 0)

    sa_ms = []
    for m in range(M):
        s_wide = s_s[m] @ GT
        sa_ms.append(acc[m] / (s_wide + 1e-16))
    sa = jnp.stack(sa_ms, axis=1)
    ego1 = x + sa
    y2 = ego1.reshape(BN * M, D)
    mu2 = jnp.mean(y2, axis=-1, keepdims=True)
    var2 = jnp.mean((y2 - mu2) ** 2, axis=-1, keepdims=True)
    ln2 = (y2 - mu2) / jnp.sqrt(var2 + 1e-5) * g2_r[...] + be2_r[...]
    h = jnp.maximum(ln2 @ W1_r[...] + b1_r[...], 0.0)
    h = h @ Wd_r[...] + bd_r[...]
    out = y2 + (h @ W2_r[...] + b2_r[...])
    out_ref[...] = out.reshape(BN, M, D)


def _full_spec(shape):
    nd = len(shape)
    return pl.BlockSpec(shape, lambda b, *_: (0,) * nd)


@jax.jit
def kernel(batch, ego_feature, obs_out, Wk, bk, Wq, bq, Wv1, bv1, Wv2, bv2,
           g1, be1, g2, be2, W1, b1l, Wd, bd, W2, b2l):
    pad = E_PAD - E
    obs_p = jnp.pad(obs_out, ((0, 0), (0, pad), (0, 0), (0, 0)))
    ef_p = obs_p[0]
    nf_p = obs_p[1]
    bt_p = jnp.concatenate(
        [batch, jnp.full((pad,), N, jnp.int32)], axis=0)
    rs = jnp.searchsorted(
        batch, jnp.arange(0, N + 1, BN, dtype=jnp.int32)).astype(jnp.int32)

    grid_spec = pltpu.PrefetchScalarGridSpec(
        num_scalar_prefetch=1,
        grid=(NB,),
        in_specs=[
            pl.BlockSpec((BN, M, D), lambda b, *_: (b, 0, 0)),
            pl.BlockSpec(memory_space=pl.ANY),
            pl.BlockSpec(memory_space=pl.ANY),
            pl.BlockSpec(memory_space=pl.ANY),
            _full_spec((D, D)), _full_spec((1, D)),
            _full_spec((DH, D)), _full_spec((1, D)),
            _full_spec((DH, D)), _full_spec((1, D)),
            _full_spec((DH, D)), _full_spec((1, D)),
            _full_spec((1, D)), _full_spec((1, D)),
            _full_spec((1, D)), _full_spec((1, D)),
            _full_spec((D, D)), _full_spec((1, D)),
            _full_spec((D, D)), _full_spec((1, D)),
            _full_spec((D, D)), _full_spec((1, D)),
        ],
        out_specs=pl.BlockSpec((BN, M, D), lambda b, *_: (b, 0, 0)),
        scratch_shapes=[
            pltpu.VMEM((2, CHUNK, M, DH), jnp.float32),
            pltpu.VMEM((2, CHUNK, M, DH), jnp.float32),
            pltpu.VMEM((2, CHUNK), jnp.int32),
            pltpu.VMEM((M, BN, H), jnp.float32),
            pltpu.VMEM((M, BN, H), jnp.float32),
            pltpu.VMEM((M, BN, D), jnp.float32),
            pltpu.SemaphoreType.DMA((2,)),
            pltpu.SemaphoreType.DMA((2,)),
            pltpu.SemaphoreType.DMA((2,)),
        ],
    )
    fn = pl.pallas_call(
        _layer_kernel,
        grid_spec=grid_spec,
        out_shape=jax.ShapeDtypeStruct((N, M, D), jnp.float32),
    )
    ego = ego_feature
    for i in range(2):
        ego = fn(rs, ego, ef_p, nf_p, bt_p,
                 Wq[i], bq[i].reshape(1, D),
                 Wk[i], bk[i].reshape(1, D),
                 Wv1[i], bv1[i].reshape(1, D),
                 Wv2[i], bv2[i].reshape(1, D),
                 g1[i].reshape(1, D), be1[i].reshape(1, D),
                 g2[i].reshape(1, D), be2[i].reshape(1, D),
                 W1[i], b1l[i].reshape(1, D),
                 Wd[i], bd[i].reshape(1, D),
                 W2[i], b2l[i].reshape(1, D))
    return ego


# speedup vs baseline: 4.0716x; 4.0716x over previous
import jax
import jax.numpy as jnp
from jax import lax
from jax.experimental import pallas as pl
from jax.experimental.pallas import tpu as pltpu

N = 10000
E = 160000
M = 6
H = 8
D = 128
DH = D // 2
HD = D // H
MD = M * D
MDH = M * DH
MH = M * H

BN = 80
NB = N // BN
CHUNK = 512
E_PAD = ((E + CHUNK - 1) // CHUNK) * CHUNK


def _ln(x, Sm_r, Sw_r, g, bb):
    mu = (x @ Sm_r) @ Sw_r
    d = x - mu
    var = ((d * d) @ Sm_r) @ Sw_r
    return d / jnp.sqrt(var + 1e-5) * g + bb


def _layer_kernel(rs_ref, ego_ref, ef_hbm, nf_hbm, bt_hbm,
                  WqB_r, bqw_r, WkB_r, bkw_r, Wv1B_r, Wv2B_r, bvw_r,
                  g1w_r, be1w_r, g2w_r, be2w_r,
                  W1B_r, b1w_r, WdB_r, bdw_r, W2B_r, b2w_r,
                  Gm_r, Sm_r, Sw_r,
                  out_ref,
                  ef_v, nf_v, bt_v, m_s, s_s, acc, sem_ef, sem_nf, sem_bt):
    b = pl.program_id(0)
    n0 = b * BN
    s0 = rs_ref[b]
    s1 = rs_ref[b + 1]
    s0_al = (s0 // CHUNK) * CHUNK
    nchunks = (s1 - s0_al + CHUNK - 1) // CHUNK

    Gm = Gm_r[...]

    x = ego_ref[...]
    ln1 = _ln(x, Sm_r[...], Sw_r[...], g1w_r[...], be1w_r[...])
    qr = ln1 @ WqB_r[...] + bqw_r[...]

    m_s[...] = jnp.full((BN, MH), -1e9, jnp.float32)
    s_s[...] = jnp.zeros((BN, MH), jnp.float32)
    acc[...] = jnp.zeros((BN, MD), jnp.float32)

    col = lax.broadcasted_iota(jnp.int32, (CHUNK, BN), 1)
    row1 = lax.broadcasted_iota(jnp.int32, (CHUNK, 1), 0)

    def start(c, slot):
        off = s0_al + c * CHUNK
        pltpu.make_async_copy(
            ef_hbm.at[pl.ds(off, CHUNK)], ef_v.at[slot], sem_ef.at[slot]
        ).start()
        pltpu.make_async_copy(
            nf_hbm.at[pl.ds(off, CHUNK)], nf_v.at[slot], sem_nf.at[slot]
        ).start()
        pltpu.make_async_copy(
            bt_hbm.at[pl.ds(off, CHUNK)], bt_v.at[slot], sem_bt.at[slot]
        ).start()

    def wait(c, slot):
        off = s0_al + c * CHUNK
        pltpu.make_async_copy(
            ef_hbm.at[pl.ds(off, CHUNK)], ef_v.at[slot], sem_ef.at[slot]
        ).wait()
        pltpu.make_async_copy(
            nf_hbm.at[pl.ds(off, CHUNK)], nf_v.at[slot], sem_nf.at[slot]
        ).wait()
        pltpu.make_async_copy(
            bt_hbm.at[pl.ds(off, CHUNK)], bt_v.at[slot], sem_bt.at[slot]
        ).wait()

    def compute(c, slot):
        off = s0_al + c * CHUNK
        ef2 = ef_v[slot]
        nf2 = nf_v[slot]
        bt = bt_v[slot].reshape(CHUNK, 1)
        li = bt - n0
        g = off + row1
        valid = (g >= s0) & (g < s1)
        onehot = jnp.where((li == col) & valid, 1.0, 0.0)

        k = ef2 @ WkB_r[...] + bkw_r[...]
        qe = onehot @ qr
        lg = (k * qe) @ Gm
        lgm = jnp.where(valid, lg, -1e9)
        cand = jnp.max(
            jnp.where(onehot[:, :, None] > 0.0, lgm[:, None, :], -1e9),
            axis=0)
        m_old = m_s[...]
        m_new = jnp.maximum(m_old, cand)
        m_s[...] = m_new
        r = jnp.exp(m_old - m_new)
        e = jnp.where(valid, jnp.exp(lg - onehot @ m_new), 0.0)
        s_s[...] = s_s[...] * r + lax.dot_general(
            onehot, e, (((0,), (0,)), ((), ())))
        v = ef2 @ Wv1B_r[...] + nf2 @ Wv2B_r[...] + bvw_r[...]
        ew = lax.dot_general(e, Gm, (((1,), (1,)), ((), ())))
        rw = lax.dot_general(r, Gm, (((1,), (1,)), ((), ())))
        acc[...] = acc[...] * rw + lax.dot_general(
            onehot, v * ew, (((0,), (0,)), ((), ())))

    @pl.when(nchunks > 0)
    def _prologue():
        start(0, 0)

    def body2(i, carry):
        c0 = 2 * i

        @pl.when(c0 + 1 < nchunks)
        def _():
            start(c0 + 1, 1)

        wait(c0, 0)
        compute(c0, 0)

        @pl.when(c0 + 2 < nchunks)
        def _():
            start(c0 + 2, 0)

        @pl.when(c0 + 1 < nchunks)
        def _():
            wait(c0 + 1, 1)
            compute(c0 + 1, 1)

        return carry

    lax.fori_loop(0, (nchunks + 1) // 2, body2, 0)

    sw = lax.dot_general(s_s[...], Gm, (((1,), (1,)), ((), ())))
    sa = acc[...] / (sw + 1e-16)
    ego1 = x + sa
    ln2 = _ln(ego1, Sm_r[...], Sw_r[...], g2w_r[...], be2w_r[...])
    h = jnp.maximum(ln2 @ W1B_r[...] + b1w_r[...], 0.0)
    h = h @ WdB_r[...] + bdw_r[...]
    out = ego1 + (h @ W2B_r[...] + b2w_r[...])
    out_ref[...] = out


def _full_spec(shape):
    nd = len(shape)
    return pl.BlockSpec(shape, lambda b, *_: (0,) * nd)


def _tile(v):
    return jnp.tile(v.reshape(1, D), (1, M))


@jax.jit
def kernel(batch, ego_feature, obs_out, Wk, bk, Wq, bq, Wv1, bv1, Wv2, bv2,
           g1, be1, g2, be2, W1, b1l, Wd, bd, W2, b2l):
    pad = E_PAD - E
    obs_p = jnp.pad(obs_out, ((0, 0), (0, pad), (0, 0), (0, 0)))
    ef_p = obs_p[0].reshape(E_PAD, MDH)
    nf_p = obs_p[1].reshape(E_PAD, MDH)
    bt_p = jnp.concatenate(
        [batch, jnp.full((pad,), N, jnp.int32)], axis=0)
    rs = jnp.searchsorted(
        batch, jnp.arange(0, N + 1, BN, dtype=jnp.int32)).astype(jnp.int32)

    eye = jnp.eye(M, dtype=jnp.float32)
    lane = jnp.arange(MD)
    head = (lane // D) * H + (lane % D) // HD
    Gm = (head[:, None] == jnp.arange(MH)[None, :]).astype(jnp.float32)
    seg = lane // D
    Sm = (seg[:, None] == jnp.arange(M)[None, :]).astype(jnp.float32) / D
    Sw = (jnp.arange(M)[:, None] == seg[None, :]).astype(jnp.float32)

    grid_spec = pltpu.PrefetchScalarGridSpec(
        num_scalar_prefetch=1,
        grid=(NB,),
        in_specs=[
            pl.BlockSpec((BN, MD), lambda b, *_: (b, 0)),
            pl.BlockSpec(memory_space=pl.ANY),
            pl.BlockSpec(memory_space=pl.ANY),
            pl.BlockSpec(memory_space=pl.ANY),
            _full_spec((MD, MD)), _full_spec((1, MD)),
            _full_spec((MDH, MD)), _full_spec((1, MD)),
            _full_spec((MDH, MD)), _full_spec((MDH, MD)),
            _full_spec((1, MD)),
            _full_spec((1, MD)), _full_spec((1, MD)),
            _full_spec((1, MD)), _full_spec((1, MD)),
            _full_spec((MD, MD)), _full_spec((1, MD)),
            _full_spec((MD, MD)), _full_spec((1, MD)),
            _full_spec((MD, MD)), _full_spec((1, MD)),
            _full_spec((MD, MH)),
            _full_spec((MD, M)), _full_spec((M, MD)),
        ],
        out_specs=pl.BlockSpec((BN, MD), lambda b, *_: (b, 0)),
        scratch_shapes=[
            pltpu.VMEM((2, CHUNK, MDH), jnp.float32),
            pltpu.VMEM((2, CHUNK, MDH), jnp.float32),
            pltpu.VMEM((2, CHUNK), jnp.int32),
            pltpu.VMEM((BN, MH), jnp.float32),
            pltpu.VMEM((BN, MH), jnp.float32),
            pltpu.VMEM((BN, MD), jnp.float32),
            pltpu.SemaphoreType.DMA((2,)),
            pltpu.SemaphoreType.DMA((2,)),
            pltpu.SemaphoreType.DMA((2,)),
        ],
    )
    fn = pl.pallas_call(
        _layer_kernel,
        grid_spec=grid_spec,
        out_shape=jax.ShapeDtypeStruct((N, MD), jnp.float32),
    )
    ego = ego_feature.reshape(N, MD)
    for i in range(2):
        ego = fn(rs, ego, ef_p, nf_p, bt_p,
                 jnp.kron(eye, Wq[i]), _tile(bq[i]),
                 jnp.kron(eye, Wk[i]), _tile(bk[i]),
                 jnp.kron(eye, Wv1[i]), jnp.kron(eye, Wv2[i]),
                 _tile(bv1[i] + bv2[i]),
                 _tile(g1[i]), _tile(be1[i]),
                 _tile(g2[i]), _tile(be2[i]),
                 jnp.kron(eye, W1[i]), _tile(b1l[i]),
                 jnp.kron(eye, Wd[i]), _tile(bd[i]),
                 jnp.kron(eye, W2[i]), _tile(b2l[i]),
                 Gm, Sm, Sw)
    return ego.reshape(N, M, D)
